# baseline (speedup 1.0000x reference)
import jax
import jax.numpy as jnp
from jax import lax
from jax.experimental import pallas as pl
from jax.experimental.pallas import tpu as pltpu, tpu_sc as plsc

VOCAB = 1000000
D = 64
B = 4096
S = 50
N = B * S

_info = plsc.get_sparse_core_info()
NC, NS = _info.num_cores, _info.num_subcores
NW = NC * NS
PER_W = N // NW
CHUNK = 128
NCHUNK = PER_W // CHUNK


def _body(idx_hbm, table_hbm, out_hbm, idx_v, rows_v, gsem):
    wid = lax.axis_index("s") * NC + lax.axis_index("c")
    base = wid * PER_W
    pltpu.sync_copy(idx_hbm.at[wid], idx_v)

    def step(j, carry):
        pltpu.async_copy(table_hbm.at[idx_v.at[j]], rows_v, gsem).wait()
        pltpu.sync_copy(rows_v, out_hbm.at[pl.ds(base + j * CHUNK, CHUNK)])
        return carry

    lax.fori_loop(0, NCHUNK, step, 0)


def kernel(x, table):
    idx = x.reshape(NW, NCHUNK, CHUNK).astype(jnp.int32)
    mesh = plsc.VectorSubcoreMesh(core_axis_name="c", subcore_axis_name="s")
    out = pl.kernel(
        _body,
        out_type=jax.ShapeDtypeStruct((N, D), jnp.float32),
        mesh=mesh,
        scratch_types=[
            pltpu.VMEM((NCHUNK, CHUNK), jnp.int32),
            pltpu.VMEM((CHUNK, D), jnp.float32),
            pltpu.SemaphoreType.DMA,
        ],
        compiler_params=pltpu.CompilerParams(use_tc_tiling_on_sc=False),
    )(idx, table)
    return out.reshape(B, S, D)


# speedup vs baseline: 1.0476x; 1.0476x over previous
import jax
import jax.numpy as jnp
from jax import lax
from jax.experimental import pallas as pl
from jax.experimental.pallas import tpu as pltpu, tpu_sc as plsc

VOCAB = 1000000
D = 64
B = 4096
S = 50
N = B * S

_info = plsc.get_sparse_core_info()
NC, NS = _info.num_cores, _info.num_subcores
NW = NC * NS
PER_W = N // NW
CHUNK = 128
NCHUNK = PER_W // CHUNK
K = 5
NG = NCHUNK // K


def _body(idx_hbm, table_hbm, out_hbm, idx_v, rows_a, rows_b, gsa, gsb, osa, osb):
    wid = lax.axis_index("s") * NC + lax.axis_index("c")
    base = wid * PER_W
    pltpu.sync_copy(idx_hbm.at[wid], idx_v)

    def fire_gathers(g, bufs, sem):
        for b in range(K):
            pltpu.async_copy(table_hbm.at[idx_v.at[g * K + b]], bufs.at[b], sem)

    def drain_gathers(bufs, sem):
        for b in range(K):
            pltpu.make_async_copy(table_hbm.at[idx_v.at[0]], bufs.at[b], sem).wait()

    def fire_outs(g, bufs, sem):
        for b in range(K):
            pltpu.async_copy(
                bufs.at[b], out_hbm.at[pl.ds(base + (g * K + b) * CHUNK, CHUNK)], sem
            )

    def drain_outs(g, bufs, sem):
        for b in range(K):
            pltpu.make_async_copy(
                bufs.at[b], out_hbm.at[pl.ds(base + (g * K + b) * CHUNK, CHUNK)], sem
            ).wait()

    fire_gathers(0, rows_a, gsa)
    fire_gathers(1, rows_b, gsb)

    def steady(m, carry):
        ga = 2 * m
        gb = 2 * m + 1
        drain_gathers(rows_a, gsa)
        fire_outs(ga, rows_a, osa)
        drain_outs(ga, rows_a, osa)
        fire_gathers(ga + 2, rows_a, gsa)
        drain_gathers(rows_b, gsb)
        fire_outs(gb, rows_b, osb)
        drain_outs(gb, rows_b, osb)
        fire_gathers(gb + 2, rows_b, gsb)
        return carry

    lax.fori_loop(0, (NG - 2) // 2, steady, 0)

    drain_gathers(rows_a, gsa)
    fire_outs(NG - 2, rows_a, osa)
    drain_gathers(rows_b, gsb)
    fire_outs(NG - 1, rows_b, osb)
    drain_outs(NG - 2, rows_a, osa)
    drain_outs(NG - 1, rows_b, osb)


def kernel(x, table):
    idx = x.reshape(NW, NCHUNK, CHUNK).astype(jnp.int32)
    mesh = plsc.VectorSubcoreMesh(core_axis_name="c", subcore_axis_name="s")
    out = pl.kernel(
        _body,
        out_type=jax.ShapeDtypeStruct((N, D), jnp.float32),
        mesh=mesh,
        scratch_types=[
            pltpu.VMEM((NCHUNK, CHUNK), jnp.int32),
            pltpu.VMEM((K, CHUNK, D), jnp.float32),
            pltpu.VMEM((K, CHUNK, D), jnp.float32),
            pltpu.SemaphoreType.DMA,
            pltpu.SemaphoreType.DMA,
            pltpu.SemaphoreType.DMA,
            pltpu.SemaphoreType.DMA,
        ],
        compiler_params=pltpu.CompilerParams(use_tc_tiling_on_sc=False),
    )(idx, table)
    return out.reshape(B, S, D)
